# Initial kernel scaffold; baseline (speedup 1.0000x reference)
import jax
import jax.numpy as jnp
from jax.experimental import pallas as pl


def kernel(x, edge_attr, edge_index, batch, Wc1, bc1, Wc2, bc2, Wl0, bl0, Wl1, bl1, Wl2, bl2):
    raise NotImplementedError("write your pallas kernel here")


# speedup vs baseline: 14.8960x; 14.8960x over previous
import functools

import jax
import jax.numpy as jnp
from jax import lax
from jax.experimental import pallas as pl
from jax.experimental.pallas import tpu as pltpu
from jax.experimental.pallas import tpu_sc as plsc

N = 10000
E = 320000
D = 128
G = 64
OUTD = 32

NC = 2
NS = 16
NW = NC * NS
CH = 80
EPT = E // NW
NROW = E // CH
RPT = NROW // NW
RSTAGE = 25
NSTAGE = RPT // RSTAGE
RPS = N // NS
ZR = 125
BN = 2000
NB = N // BN

_mesh = plsc.VectorSubcoreMesh(
    core_axis_name="c", subcore_axis_name="s", num_cores=NC, num_subcores=NS)
_sc_params = pltpu.CompilerParams(
    use_tc_tiling_on_sc=False, needs_layout_passes=False)


def _deg_body(col_hbm, ew_hbm, pdeg_hbm, cbuf, wbuf, deg_v):
    cid = lax.axis_index("c")
    sid = lax.axis_index("s")
    wid = cid * NS + sid

    def zero(i, _):
        deg_v[pl.ds(i * 16, 16)] = jnp.zeros((16,), jnp.float32)
        return _

    lax.fori_loop(0, N // 16, zero, 0, unroll=8)

    for st in range(NSTAGE):
        r0 = wid * RPT + st * RSTAGE
        pltpu.sync_copy(col_hbm.at[pl.ds(r0, RSTAGE)], cbuf)
        pltpu.sync_copy(ew_hbm.at[pl.ds(r0, RSTAGE)], wbuf)

        def body(j, _):
            for q in range(CH // 16):
                c = cbuf[j, pl.ds(q * 16, 16)]
                w = wbuf[j, pl.ds(q * 16, 16)]
                plsc.addupdate_scatter(deg_v, [c], w)
            return _

        lax.fori_loop(0, RSTAGE, body, 0)

    for kb in range(NB):
        pltpu.sync_copy(deg_v.at[pl.ds(kb * BN, BN)], pdeg_hbm.at[kb, wid])


_deg_kernel = functools.partial(
    pl.kernel,
    out_type=jax.ShapeDtypeStruct((NB, NW, BN), jnp.float32),
    mesh=_mesh,
    compiler_params=_sc_params,
    scratch_types=[
        pltpu.VMEM((RSTAGE, CH), jnp.int32),
        pltpu.VMEM((RSTAGE, CH), jnp.float32),
        pltpu.VMEM((N,), jnp.float32),
    ],
)(_deg_body)


def _edge_body(y_hbm, row_hbm, col_hbm, ew_hbm, out_hbm,
               rbuf, cbuf, wbuf, gbuf, zbuf, acc, sem):
    cid = lax.axis_index("c")
    sid = lax.axis_index("s")
    wid = cid * NS + sid

    def zero(i, _):
        zbuf[i // 8, pl.ds((i % 8) * 16, 16)] = jnp.zeros((16,), jnp.float32)
        return _

    lax.fori_loop(0, ZR * 8, zero, 0, unroll=8)
    for k in range(RPS // ZR):
        pltpu.sync_copy(zbuf, acc.at[pl.ds(sid * RPS + k * ZR, ZR)])
    plsc.subcore_barrier()

    for st in range(NSTAGE):
        r0 = wid * RPT + st * RSTAGE
        pltpu.sync_copy(row_hbm.at[pl.ds(r0, RSTAGE)], rbuf)
        pltpu.sync_copy(col_hbm.at[pl.ds(r0, RSTAGE)], cbuf)
        pltpu.sync_copy(ew_hbm.at[pl.ds(r0, RSTAGE)], wbuf)

        def chunk(j, _):
            pltpu.async_copy(y_hbm.at[rbuf.at[j]], gbuf, sem).wait()
            for e in range(CH):
                w = plsc.load_gather(
                    wbuf,
                    [jnp.full((16,), j, jnp.int32),
                     jnp.full((16,), e, jnp.int32)])
                for q in range(D // 16):
                    sl = pl.ds(q * 16, 16)
                    gbuf[e, sl] = gbuf[e, sl] * w
            pltpu.sync_copy(gbuf, acc.at[cbuf.at[j]], add=True)
            return _

        lax.fori_loop(0, RSTAGE, chunk, 0)

    plsc.subcore_barrier()
    for k in range(RPS // ZR):
        r0 = sid * RPS + k * ZR
        pltpu.sync_copy(acc.at[pl.ds(r0, ZR)], zbuf)
        pltpu.sync_copy(zbuf, out_hbm.at[cid, pl.ds(r0, ZR)])


_edge_kernel = functools.partial(
    pl.kernel,
    out_type=jax.ShapeDtypeStruct((NC, N, D), jnp.float32),
    mesh=_mesh,
    compiler_params=_sc_params,
    scratch_types=[
        pltpu.VMEM((RSTAGE, CH), jnp.int32),
        pltpu.VMEM((RSTAGE, CH), jnp.int32),
        pltpu.VMEM((RSTAGE, CH), jnp.float32),
        pltpu.VMEM((CH, D), jnp.float32),
        pltpu.VMEM((ZR, D), jnp.float32),
        pltpu.VMEM_SHARED((N, D), jnp.float32),
        pltpu.SemaphoreType.DMA,
    ],
)(_edge_body)


def _dinv_block(pdeg_ref):
    deg = 1.0 + jnp.sum(pdeg_ref[0], axis=0)
    return lax.rsqrt(jnp.maximum(deg, 1e-12))


def _prep_body(x_ref, w_ref, pdeg_ref, y_ref):
    dinv = _dinv_block(pdeg_ref)
    xw = jnp.dot(x_ref[...], w_ref[...], preferred_element_type=jnp.float32)
    y_ref[...] = dinv[:, None] * xw


def _tc_prep(x, Wc1, pdeg):
    return pl.pallas_call(
        _prep_body,
        grid=(NB,),
        in_specs=[
            pl.BlockSpec((BN, D), lambda i: (i, 0)),
            pl.BlockSpec((D, D), lambda i: (0, 0)),
            pl.BlockSpec((1, NW, BN), lambda i: (i, 0, 0)),
        ],
        out_specs=pl.BlockSpec((BN, D), lambda i: (i, 0)),
        out_shape=jax.ShapeDtypeStruct((N, D), jnp.float32),
    )(x, Wc1, pdeg)


def _mid_body(acc_ref, y_ref, pdeg_ref, b_ref, w_ref, y2_ref):
    dinv = _dinv_block(pdeg_ref)
    t = acc_ref[0] + acc_ref[1] + y_ref[...]
    h = jnp.maximum(dinv[:, None] * t + b_ref[...], 0.0)
    hw = jnp.dot(h, w_ref[...], preferred_element_type=jnp.float32)
    y2_ref[...] = dinv[:, None] * hw


def _tc_mid(acc, y, pdeg, b, W):
    return pl.pallas_call(
        _mid_body,
        grid=(NB,),
        in_specs=[
            pl.BlockSpec((NC, BN, D), lambda i: (0, i, 0)),
            pl.BlockSpec((BN, D), lambda i: (i, 0)),
            pl.BlockSpec((1, NW, BN), lambda i: (i, 0, 0)),
            pl.BlockSpec((1, D), lambda i: (0, 0)),
            pl.BlockSpec((D, D), lambda i: (0, 0)),
        ],
        out_specs=pl.BlockSpec((BN, D), lambda i: (i, 0)),
        out_shape=jax.ShapeDtypeStruct((N, D), jnp.float32),
    )(acc, y, pdeg, b.reshape(1, D), W)


def _final_body(acc_ref, y_ref, pdeg_ref, b_ref, batch_ref,
                w0_ref, b0_ref, w1_ref, b1_ref, w2_ref, b2_ref,
                out_ref, sums, cnts):
    i = pl.program_id(0)
    dinv = _dinv_block(pdeg_ref)
    t = acc_ref[0] + acc_ref[1] + y_ref[...]
    h = jnp.maximum(dinv[:, None] * t + b_ref[...], 0.0)

    seg = lax.broadcasted_iota(jnp.int32, (G, BN), 0)
    oh = (batch_ref[0, 0, :][None, :] == seg).astype(jnp.float32)

    @pl.when(i == 0)
    def _():
        sums[...] = jnp.zeros((G, D), jnp.float32)
        cnts[...] = jnp.zeros((G, D), jnp.float32)

    sums[...] += jnp.dot(oh, h, preferred_element_type=jnp.float32)
    cnts[...] += jnp.dot(oh, jnp.ones((BN, D), jnp.float32),
                         preferred_element_type=jnp.float32)

    @pl.when(i == NB - 1)
    def _():
        g = sums[...] / jnp.maximum(cnts[...], 1.0)
        g = jnp.maximum(
            jnp.dot(g, w0_ref[...], preferred_element_type=jnp.float32)
            + b0_ref[...], 0.0)
        g = jnp.maximum(
            jnp.dot(g, w1_ref[...], preferred_element_type=jnp.float32)
            + b1_ref[...], 0.0)
        out_ref[...] = (
            jnp.dot(g, w2_ref[...], preferred_element_type=jnp.float32)
            + b2_ref[...])


def _tc_final(acc, y, pdeg, b, batch3, Wl0, bl0, Wl1, bl1, Wl2, bl2):
    return pl.pallas_call(
        _final_body,
        grid=(NB,),
        in_specs=[
            pl.BlockSpec((NC, BN, D), lambda i: (0, i, 0)),
            pl.BlockSpec((BN, D), lambda i: (i, 0)),
            pl.BlockSpec((1, NW, BN), lambda i: (i, 0, 0)),
            pl.BlockSpec((1, D), lambda i: (0, 0)),
            pl.BlockSpec((1, 1, BN), lambda i: (i, 0, 0)),
            pl.BlockSpec((D, D), lambda i: (0, 0)),
            pl.BlockSpec((1, D), lambda i: (0, 0)),
            pl.BlockSpec((D, D), lambda i: (0, 0)),
            pl.BlockSpec((1, D), lambda i: (0, 0)),
            pl.BlockSpec((D, OUTD), lambda i: (0, 0)),
            pl.BlockSpec((1, OUTD), lambda i: (0, 0)),
        ],
        out_specs=pl.BlockSpec((G, OUTD), lambda i: (0, 0)),
        out_shape=jax.ShapeDtypeStruct((G, OUTD), jnp.float32),
        scratch_shapes=[
            pltpu.VMEM((G, D), jnp.float32),
            pltpu.VMEM((G, D), jnp.float32),
        ],
    )(acc, y, pdeg, b.reshape(1, D), batch3,
      Wl0, bl0.reshape(1, D), Wl1, bl1.reshape(1, D),
      Wl2, bl2.reshape(1, OUTD))


@jax.jit
def kernel(x, edge_attr, edge_index, batch,
           Wc1, bc1, Wc2, bc2, Wl0, bl0, Wl1, bl1, Wl2, bl2):
    row2 = edge_index[0].astype(jnp.int32).reshape(NROW, CH)
    col2 = edge_index[1].astype(jnp.int32).reshape(NROW, CH)
    ew2 = edge_attr.reshape(NROW, CH)
    batch3 = batch.astype(jnp.int32).reshape(NB, 1, BN)

    pdeg = _deg_kernel(col2, ew2)
    y1 = _tc_prep(x, Wc1, pdeg)
    acc1 = _edge_kernel(y1, row2, col2, ew2)
    y2 = _tc_mid(acc1, y1, pdeg, bc1, Wc2)
    acc2 = _edge_kernel(y2, row2, col2, ew2)
    return _tc_final(acc2, y2, pdeg, bc2, batch3,
                     Wl0, bl0, Wl1, bl1, Wl2, bl2)
